# Initial kernel scaffold; baseline (speedup 1.0000x reference)
import jax
import jax.numpy as jnp
from jax.experimental import pallas as pl


def kernel(inputs, indexes, features, labels):
    raise NotImplementedError("write your pallas kernel here")


# speedup vs baseline: 8.0964x; 8.0964x over previous
import jax
import jax.numpy as jnp
from jax.experimental import pallas as pl

_TEMP = 0.05
_C = 1000
_C_PAD = 1024
_NB = 2000


def _seg_body(lab_ref, feat_ref, cs_ref, cnt_ref):
    g = pl.program_id(0)
    fb = feat_ref[...]
    ss = jnp.sum(fb * fb, axis=1, keepdims=True)
    rn = jax.lax.rsqrt(jnp.maximum(ss, 1e-24))
    fn = (fb * rn).astype(jnp.bfloat16)
    lab = lab_ref[0]
    cio = jax.lax.broadcasted_iota(jnp.int32, (_C_PAD, _NB), 0)
    oh = (cio == lab).astype(jnp.bfloat16)
    csb = jax.lax.dot_general(oh, fn, (((1,), (0,)), ((), ())),
                              preferred_element_type=jnp.float32)
    ones8 = jnp.ones((8, _NB), jnp.bfloat16)
    cntb = jax.lax.dot_general(ones8, oh, (((1,), (1,)), ((), ())),
                               preferred_element_type=jnp.float32)

    @pl.when(g == 0)
    def _init():
        cs_ref[...] = jnp.zeros_like(cs_ref)
        cnt_ref[...] = jnp.zeros_like(cnt_ref)

    cs_ref[...] += csb
    cnt_ref[...] += cntb


def _epi_body(in_ref, idx_ref, lab2_ref, cs_ref, cnt_ref, out_ref):
    b = in_ref.shape[0]
    u = in_ref[...]
    ss = jnp.sum(u * u, axis=1, keepdims=True)
    un = u * jax.lax.rsqrt(jnp.maximum(ss, 1e-24))
    logits = jax.lax.dot_general(un, cs_ref[...], (((1,), (1,)), ((), ())),
                                 preferred_element_type=jnp.float32)
    cntrow = cnt_ref[0:1, :]
    mask = cntrow > 0.0
    denom = jnp.where(mask, cntrow, 1.0)
    sim = logits / (_TEMP * denom)
    exps = jnp.exp(sim) * mask.astype(jnp.float32)
    sums = jnp.sum(exps, axis=1, keepdims=True) + 1e-6
    logp = jnp.log(exps / sums + 1e-6)
    idx = idx_ref[...]
    q = idx // _C
    r = idx - q * _C
    l2 = lab2_ref[...].astype(jnp.float32)
    qio = jax.lax.broadcasted_iota(jnp.int32, (b, l2.shape[0]), 1)
    ohq = (qio == q).astype(jnp.float32)
    rowsel = jax.lax.dot_general(ohq, l2, (((1,), (0,)), ((), ())),
                                 preferred_element_type=jnp.float32)
    rio = jax.lax.broadcasted_iota(jnp.int32, (b, _C), 1)
    ohr = (rio == r).astype(jnp.float32)
    tcol = jnp.sum(rowsel * ohr, axis=1, keepdims=True)
    cio = jax.lax.broadcasted_iota(jnp.int32, (b, _C_PAD), 1)
    oht = (cio == tcol.astype(jnp.int32)).astype(jnp.float32)
    loss = -jnp.sum(logp * oht) / float(b)
    out_ref[...] = jnp.full((1, 1), loss, jnp.float32)


def kernel(inputs, indexes, features, labels):
    n, f = features.shape
    b = inputs.shape[0]
    k = n // _NB
    labels_b = labels.reshape(k, 1, _NB)

    cs, cnt = pl.pallas_call(
        _seg_body,
        grid=(k,),
        in_specs=[
            pl.BlockSpec((1, 1, _NB), lambda g: (g, 0, 0)),
            pl.BlockSpec((_NB, f), lambda g: (g, 0)),
        ],
        out_specs=[
            pl.BlockSpec((_C_PAD, f), lambda g: (0, 0)),
            pl.BlockSpec((8, _C_PAD), lambda g: (0, 0)),
        ],
        out_shape=[
            jax.ShapeDtypeStruct((_C_PAD, f), jnp.float32),
            jax.ShapeDtypeStruct((8, _C_PAD), jnp.float32),
        ],
    )(labels_b, features)

    loss = pl.pallas_call(
        _epi_body,
        in_specs=[pl.BlockSpec(s.shape, lambda: (0,) * len(s.shape))
                  for s in (inputs, indexes.reshape(b, 1),
                            labels.reshape(n // _C, _C), cs, cnt)],
        out_specs=pl.BlockSpec((1, 1), lambda: (0, 0)),
        out_shape=jax.ShapeDtypeStruct((1, 1), jnp.float32),
    )(inputs, indexes.reshape(b, 1), labels.reshape(n // _C, _C), cs, cnt)

    return loss[0, 0]


# speedup vs baseline: 8.5585x; 1.0571x over previous
import jax
import jax.numpy as jnp
from jax.experimental import pallas as pl

_TEMP = 0.05
_C = 1000
_C_PAD = 1024
_NB = 4000


def _seg_body(lab_ref, feat_ref, cs_ref, cnt_ref):
    g = pl.program_id(0)
    fb = feat_ref[...]
    ss = jnp.sum(fb * fb, axis=1, keepdims=True)
    rn = jax.lax.rsqrt(jnp.maximum(ss, 1e-24))
    fn = (fb * rn).astype(jnp.bfloat16)
    lab = lab_ref[0]
    cio = jax.lax.broadcasted_iota(jnp.int32, (_C_PAD, _NB), 0)
    oh = (cio == lab).astype(jnp.bfloat16)
    csb = jax.lax.dot_general(oh, fn, (((1,), (0,)), ((), ())),
                              preferred_element_type=jnp.float32)
    ones8 = jnp.ones((8, _NB), jnp.bfloat16)
    cntb = jax.lax.dot_general(ones8, oh, (((1,), (1,)), ((), ())),
                               preferred_element_type=jnp.float32)

    @pl.when(g == 0)
    def _init():
        cs_ref[...] = jnp.zeros_like(cs_ref)
        cnt_ref[...] = jnp.zeros_like(cnt_ref)

    cs_ref[...] += csb
    cnt_ref[...] += cntb


def _epi_body(in_ref, idx_ref, lab2_ref, cs_ref, cnt_ref, out_ref):
    b = in_ref.shape[0]
    u = in_ref[...]
    ss = jnp.sum(u * u, axis=1, keepdims=True)
    un = u * jax.lax.rsqrt(jnp.maximum(ss, 1e-24))
    logits = jax.lax.dot_general(un, cs_ref[...], (((1,), (1,)), ((), ())),
                                 preferred_element_type=jnp.float32)
    cntrow = cnt_ref[0:1, :]
    mask = cntrow > 0.0
    denom = jnp.where(mask, cntrow, 1.0)
    sim = logits / (_TEMP * denom)
    exps = jnp.exp(sim) * mask.astype(jnp.float32)
    sums = jnp.sum(exps, axis=1, keepdims=True) + 1e-6
    logp = jnp.log(exps / sums + 1e-6)
    idx = idx_ref[...]
    q = idx // _C
    r = idx - q * _C
    l2 = lab2_ref[...].astype(jnp.float32)
    qio = jax.lax.broadcasted_iota(jnp.int32, (b, l2.shape[0]), 1)
    ohq = (qio == q).astype(jnp.float32)
    rowsel = jax.lax.dot_general(ohq, l2, (((1,), (0,)), ((), ())),
                                 preferred_element_type=jnp.float32)
    rio = jax.lax.broadcasted_iota(jnp.int32, (b, _C), 1)
    ohr = (rio == r).astype(jnp.float32)
    tcol = jnp.sum(rowsel * ohr, axis=1, keepdims=True)
    cio = jax.lax.broadcasted_iota(jnp.int32, (b, _C_PAD), 1)
    oht = (cio == tcol.astype(jnp.int32)).astype(jnp.float32)
    loss = -jnp.sum(logp * oht) / float(b)
    out_ref[...] = jnp.full((1, 1), loss, jnp.float32)


def kernel(inputs, indexes, features, labels):
    n, f = features.shape
    b = inputs.shape[0]
    k = n // _NB
    labels_b = labels.reshape(k, 1, _NB)

    cs, cnt = pl.pallas_call(
        _seg_body,
        grid=(k,),
        in_specs=[
            pl.BlockSpec((1, 1, _NB), lambda g: (g, 0, 0)),
            pl.BlockSpec((_NB, f), lambda g: (g, 0)),
        ],
        out_specs=[
            pl.BlockSpec((_C_PAD, f), lambda g: (0, 0)),
            pl.BlockSpec((8, _C_PAD), lambda g: (0, 0)),
        ],
        out_shape=[
            jax.ShapeDtypeStruct((_C_PAD, f), jnp.float32),
            jax.ShapeDtypeStruct((8, _C_PAD), jnp.float32),
        ],
    )(labels_b, features)

    loss = pl.pallas_call(
        _epi_body,
        in_specs=[pl.BlockSpec(s.shape, lambda: (0,) * len(s.shape))
                  for s in (inputs, indexes.reshape(b, 1),
                            labels.reshape(n // _C, _C), cs, cnt)],
        out_specs=pl.BlockSpec((1, 1), lambda: (0, 0)),
        out_shape=jax.ShapeDtypeStruct((1, 1), jnp.float32),
    )(inputs, indexes.reshape(b, 1), labels.reshape(n // _C, _C), cs, cnt)

    return loss[0, 0]
